# baseline (speedup 1.0000x reference)
import functools

import jax
import jax.numpy as jnp
from jax import lax
from jax.experimental import pallas as pl
from jax.experimental.pallas import tpu as pltpu
from jax.experimental.pallas import tpu_sc as plsc

VOCAB = 100000
EMB = 16
B = 16384
L = 50

NUM_CORES = 2
NUM_SUBCORES = 16
NW = NUM_CORES * NUM_SUBCORES
SAMPLES_PER_W = B // NW
GROUPS_PER_W = SAMPLES_PER_W // 16

_RW = 256
_RROWS = VOCAB * EMB // _RW
_BLK = 3200
_TW_PAD = 2 * _BLK * EMB


def _tw_body(table_ref, w_ref, b_ref, out_ref):
    w16 = jnp.broadcast_to(w_ref[...], (EMB, EMB))
    w_tile = jnp.concatenate([w16] * EMB, axis=0)
    r_div = lax.broadcasted_iota(jnp.int32, (_RW, EMB), 0) // EMB
    c_idx = lax.broadcasted_iota(jnp.int32, (_RW, EMB), 1)
    wbig = jnp.where(r_div == c_idx, w_tile, jnp.float32(0.0))
    y = jnp.dot(table_ref[...], wbig, preferred_element_type=jnp.float32)
    out_ref[...] = y + b_ref[0]


def _compute_tw(table_r, W, b):
    return pl.pallas_call(
        _tw_body,
        grid=(2,),
        in_specs=[
            pl.BlockSpec((_BLK, _RW), lambda i: (i, 0)),
            pl.BlockSpec((EMB, 1), lambda i: (0, 0)),
            pl.BlockSpec(memory_space=pltpu.SMEM),
        ],
        out_specs=pl.BlockSpec((_BLK, EMB), lambda i: (i, 0)),
        out_shape=jax.ShapeDtypeStruct((2 * _BLK, EMB), jnp.float32),
    )(table_r, W, b)


def _sc_body(tw_hbm, x_hbm, out_hbm, tw_v, x_v, out_v):
    wid = lax.axis_index("s") * NUM_CORES + lax.axis_index("c")
    base_s = wid * SAMPLES_PER_W

    pltpu.sync_copy(tw_hbm, tw_v)
    pltpu.sync_copy(x_hbm.at[pl.ds(base_s * L, SAMPLES_PER_W * L)], x_v)

    iota = lax.iota(jnp.int32, 16)
    lane_off = iota * L
    inv_l = jnp.float32(1.0 / L)
    two_p23 = jnp.float32(16777216.0)

    def group(g, carry):
        goff = lane_off + g * (16 * L)
        acc0 = jnp.zeros((16,), jnp.float32)
        acc1 = jnp.zeros((16,), jnp.float32)
        for j in range(0, L, 2):
            xi0 = plsc.load_gather(x_v, [goff + j])
            acc0 = acc0 + plsc.load_gather(tw_v, [xi0])
            xi1 = plsc.load_gather(x_v, [goff + (j + 1)])
            acc1 = acc1 + plsc.load_gather(tw_v, [xi1])
        z = (acc0 + acc1) * inv_l
        y = 1.0 / (1.0 + jnp.exp(-z))
        t = y * jnp.float32(10000.0)
        r = (t + two_p23) - two_p23
        plsc.store_scatter(out_v, [g * 16 + iota], r * jnp.float32(1e-4))
        return carry

    lax.fori_loop(0, GROUPS_PER_W, group, 0)

    pltpu.sync_copy(out_v, out_hbm.at[pl.ds(base_s, SAMPLES_PER_W)])


def _sc_gather(tw_flat, x):
    mesh = plsc.VectorSubcoreMesh(core_axis_name="c", subcore_axis_name="s")
    k = functools.partial(
        pl.kernel,
        mesh=mesh,
        out_type=jax.ShapeDtypeStruct((B,), jnp.float32),
        scratch_types=[
            pltpu.VMEM((_TW_PAD,), jnp.float32),
            pltpu.VMEM((SAMPLES_PER_W * L,), jnp.int32),
            pltpu.VMEM((SAMPLES_PER_W,), jnp.float32),
        ],
        compiler_params=pltpu.CompilerParams(needs_layout_passes=False),
    )(_sc_body)
    return k(tw_flat, x)


def kernel(x, table, W, b):
    x = x.astype(jnp.int32).reshape(B * L)
    table_r = table.reshape(_RROWS, _RW)
    tw = _compute_tw(table_r, W, b)
    out = _sc_gather(tw.reshape(_TW_PAD), x)
    return out.reshape(B, 1)
